# Initial kernel scaffold; baseline (speedup 1.0000x reference)
import jax
import jax.numpy as jnp
from jax.experimental import pallas as pl


def kernel(x, edge_index):
    raise NotImplementedError("write your pallas kernel here")


# speedup vs baseline: 6.5321x; 6.5321x over previous
import functools

import jax
import jax.numpy as jnp
from jax import lax
from jax.experimental import pallas as pl
from jax.experimental.pallas import tpu as pltpu
from jax.experimental.pallas import tpu_sc as plsc

N_NODES = 10000
N_EDGES = 320000
D = 128
NC = 2
NS = 16
CHUNK = 128

EDGES_PER_CORE = N_EDGES // NC
CHUNKS_PER_CORE = EDGES_PER_CORE // CHUNK
ROW_STRIDE = 624
ROW_SPAN = 640


def _sc_partials():
    mesh = plsc.VectorSubcoreMesh(core_axis_name="c", subcore_axis_name="s")

    @functools.partial(
        pl.kernel,
        out_type=jax.ShapeDtypeStruct((NC, N_NODES, D), jnp.float32),
        mesh=mesh,
        scratch_types=[
            pltpu.VMEM_SHARED((N_NODES, D), jnp.float32),
            pltpu.VMEM((CHUNK,), jnp.int32),
            pltpu.VMEM((CHUNK,), jnp.int32),
            pltpu.VMEM((CHUNK, D), jnp.float32),
            pltpu.SemaphoreType.DMA,
        ],
    )
    def k(x_hbm, esrc_hbm, edst_hbm, zeros_hbm, part_hbm,
          acc, src_idx, dst_idx, rows, sem):
        c = lax.axis_index("c")
        t = lax.axis_index("s")

        pltpu.sync_copy(zeros_hbm, acc.at[pl.ds(t * ROW_STRIDE, ROW_SPAN)])
        plsc.subcore_barrier()

        ebase = c * EDGES_PER_CORE

        def chunk_body(j, carry):
            g = j * NS + t
            base = ebase + g * CHUNK
            pltpu.sync_copy(esrc_hbm.at[pl.ds(base, CHUNK)], src_idx)
            pltpu.sync_copy(edst_hbm.at[pl.ds(base, CHUNK)], dst_idx)
            pltpu.async_copy(x_hbm.at[src_idx], rows, sem).wait()
            pltpu.sync_copy(rows, acc.at[dst_idx], add=True)
            return carry

        n_j = CHUNKS_PER_CORE // NS + (t < CHUNKS_PER_CORE % NS).astype(jnp.int32)
        lax.fori_loop(0, n_j, chunk_body, 0)
        plsc.subcore_barrier()

        r0 = t * ROW_STRIDE
        pltpu.sync_copy(acc.at[pl.ds(r0, ROW_SPAN)],
                        part_hbm.at[c, pl.ds(r0, ROW_SPAN)])

    return k


_sc_kernel = _sc_partials()


def _merge_body(a_ref, b_ref, o_ref):
    o_ref[...] = a_ref[...] + b_ref[...]


def kernel(x, edge_index):
    esrc = edge_index[1]
    edst = edge_index[0]
    zeros = jnp.zeros((ROW_SPAN, D), jnp.float32)
    partials = _sc_kernel(x, esrc, edst, zeros)
    blk = 1000
    out = pl.pallas_call(
        _merge_body,
        out_shape=jax.ShapeDtypeStruct((N_NODES, D), jnp.float32),
        grid=(N_NODES // blk,),
        in_specs=[pl.BlockSpec((blk, D), lambda i: (i, 0))] * 2,
        out_specs=pl.BlockSpec((blk, D), lambda i: (i, 0)),
    )(partials[0], partials[1])
    return out


# speedup vs baseline: 12.3699x; 1.8937x over previous
import functools

import jax
import jax.numpy as jnp
from jax import lax
from jax.experimental import pallas as pl
from jax.experimental.pallas import tpu as pltpu
from jax.experimental.pallas import tpu_sc as plsc

N_NODES = 10000
N_EDGES = 320000
D = 128
NC = 2
NS = 16
CHUNK = 128

EDGES_PER_CORE = N_EDGES // NC
CHUNKS_PER_CORE = EDGES_PER_CORE // CHUNK
ROW_STRIDE = 624
ROW_SPAN = 640


def _sc_partials():
    mesh = plsc.VectorSubcoreMesh(core_axis_name="c", subcore_axis_name="s")

    NBI = 4
    NBR = 2
    N_ITERS = -(-CHUNKS_PER_CORE // NS)

    @functools.partial(
        pl.kernel,
        out_type=jax.ShapeDtypeStruct((NC, N_NODES, D), jnp.float32),
        mesh=mesh,
        scratch_types=[
            pltpu.VMEM_SHARED((N_NODES, D), jnp.float32),
            pltpu.VMEM((NBI, CHUNK), jnp.int32),
            pltpu.VMEM((NBI, CHUNK), jnp.int32),
            pltpu.VMEM((NBR, CHUNK, D), jnp.float32),
            pltpu.SemaphoreType.DMA((NBI,)),
            pltpu.SemaphoreType.DMA((NBR,)),
        ],
    )
    def k(x_hbm, esrc_hbm, edst_hbm, zeros_hbm, part_hbm,
          acc, src_idx, dst_idx, rows, isem, gsem):
        c = lax.axis_index("c")
        t = lax.axis_index("s")

        pltpu.sync_copy(zeros_hbm, acc.at[pl.ds(t * ROW_STRIDE, ROW_SPAN)])

        ebase = c * EDGES_PER_CORE

        def chunk_base(j):
            g = j * NS + t
            valid = g < CHUNKS_PER_CORE
            base = ebase + jnp.where(valid, g, 0) * CHUNK
            return base, valid

        def start_idx(j, bi):
            base, _ = chunk_base(j)
            pltpu.async_copy(esrc_hbm.at[pl.ds(base, CHUNK)], src_idx.at[bi],
                             isem.at[bi])
            pltpu.async_copy(edst_hbm.at[pl.ds(base, CHUNK)], dst_idx.at[bi],
                             isem.at[bi])

        def wait_idx(j, bi):
            base, _ = chunk_base(j)
            pltpu.make_async_copy(esrc_hbm.at[pl.ds(base, CHUNK)],
                                  src_idx.at[bi], isem.at[bi]).wait()
            pltpu.make_async_copy(edst_hbm.at[pl.ds(base, CHUNK)],
                                  dst_idx.at[bi], isem.at[bi]).wait()

        def start_gather(bi, br):
            pltpu.async_copy(x_hbm.at[src_idx.at[bi]], rows.at[br], gsem.at[br])

        def wait_gather(bi, br):
            pltpu.make_async_copy(x_hbm.at[src_idx.at[bi]], rows.at[br],
                                  gsem.at[br]).wait()

        def scatter(j, bi, br):
            _, valid = chunk_base(j)

            @pl.when(valid)
            def _():
                pltpu.sync_copy(rows.at[br], acc.at[dst_idx.at[bi]], add=True)

        for p in range(NBI - 1):
            start_idx(p, p)
        plsc.subcore_barrier()
        wait_idx(0, 0)
        start_gather(0, 0)

        def loop_body(j, carry):
            bi, br = j % NBI, j % NBR
            bi1, br1 = (j + 1) % NBI, (j + 1) % NBR
            wait_idx(j + 1, bi1)
            start_gather(bi1, br1)
            wait_gather(bi, br)
            scatter(j, bi, br)
            jn = j + NBI - 1
            start_idx(jn, jn % NBI)
            return carry

        lax.fori_loop(0, N_ITERS - 1, loop_body, 0)
        jl = N_ITERS - 1
        wait_gather(jl % NBI, jl % NBR)
        scatter(jl, jl % NBI, jl % NBR)
        for p in range(NBI - 2):
            j = N_ITERS + p
            wait_idx(j, j % NBI)
        plsc.subcore_barrier()

        r0 = t * ROW_STRIDE
        pltpu.sync_copy(acc.at[pl.ds(r0, ROW_SPAN)],
                        part_hbm.at[c, pl.ds(r0, ROW_SPAN)])

    return k


_sc_kernel = _sc_partials()


def _merge_body(a_ref, b_ref, o_ref):
    o_ref[...] = a_ref[...] + b_ref[...]


def kernel(x, edge_index):
    esrc = edge_index[1]
    edst = edge_index[0]
    zeros = jnp.zeros((ROW_SPAN, D), jnp.float32)
    partials = _sc_kernel(x, esrc, edst, zeros)
    blk = 1000
    out = pl.pallas_call(
        _merge_body,
        out_shape=jax.ShapeDtypeStruct((N_NODES, D), jnp.float32),
        grid=(N_NODES // blk,),
        in_specs=[pl.BlockSpec((blk, D), lambda i: (i, 0))] * 2,
        out_specs=pl.BlockSpec((blk, D), lambda i: (i, 0)),
    )(partials[0], partials[1])
    return out


# speedup vs baseline: 14.4183x; 1.1656x over previous
import functools

import jax
import jax.numpy as jnp
from jax import lax
from jax.experimental import pallas as pl
from jax.experimental.pallas import tpu as pltpu
from jax.experimental.pallas import tpu_sc as plsc

N_NODES = 10000
N_EDGES = 320000
D = 128
NC = 2
NS = 16
CHUNK = 128

EDGES_PER_CORE = N_EDGES // NC
CHUNKS_PER_CORE = EDGES_PER_CORE // CHUNK
ROW_STRIDE = 624
ROW_SPAN = 640

NBI = 5
NBR = 2
N_ITERS = -(-CHUNKS_PER_CORE // NS)


def _sc_partials():
    mesh = plsc.VectorSubcoreMesh(core_axis_name="c", subcore_axis_name="s")

    @functools.partial(
        pl.kernel,
        out_type=(jax.ShapeDtypeStruct((N_NODES, D), jnp.float32),
                  jax.ShapeDtypeStruct((N_NODES, D), jnp.float32)),
        mesh=mesh,
        scratch_types=[
            pltpu.VMEM_SHARED((N_NODES, D), jnp.float32),
            pltpu.VMEM((NBI, CHUNK), jnp.int32),
            pltpu.VMEM((NBI, CHUNK), jnp.int32),
            pltpu.VMEM((NBR, CHUNK, D), jnp.float32),
            pltpu.SemaphoreType.DMA((NBI,)),
            pltpu.SemaphoreType.DMA((NBR,)),
            pltpu.SemaphoreType.DMA((NBR,)),
        ],
    )
    def k(x_hbm, ei_hbm, zeros_hbm, part0_hbm, part1_hbm,
          acc, src_idx, dst_idx, rows, isem, gsem, ssem):
        c = lax.axis_index("c")
        t = lax.axis_index("s")

        ebase = c * EDGES_PER_CORE

        def chunk_base(j):
            g = j * NS + t
            valid = g < CHUNKS_PER_CORE
            base = ebase + jnp.where(valid, g, 0) * CHUNK
            return base, valid

        def start_idx(j):
            bi = j % NBI
            base, _ = chunk_base(j)
            pltpu.async_copy(ei_hbm.at[1, pl.ds(base, CHUNK)], src_idx.at[bi],
                             isem.at[bi])
            pltpu.async_copy(ei_hbm.at[0, pl.ds(base, CHUNK)], dst_idx.at[bi],
                             isem.at[bi])

        def wait_idx(j):
            bi = j % NBI
            base, _ = chunk_base(j)
            pltpu.make_async_copy(ei_hbm.at[1, pl.ds(base, CHUNK)],
                                  src_idx.at[bi], isem.at[bi]).wait()
            pltpu.make_async_copy(ei_hbm.at[0, pl.ds(base, CHUNK)],
                                  dst_idx.at[bi], isem.at[bi]).wait()

        def start_gather(j):
            bi, br = j % NBI, j % NBR
            pltpu.async_copy(x_hbm.at[src_idx.at[bi]], rows.at[br], gsem.at[br])

        def wait_gather(j):
            bi, br = j % NBI, j % NBR
            pltpu.make_async_copy(x_hbm.at[src_idx.at[bi]], rows.at[br],
                                  gsem.at[br]).wait()

        def start_scatter(j):
            bi, br = j % NBI, j % NBR
            _, valid = chunk_base(j)

            @pl.when(valid)
            def _():
                pltpu.async_copy(rows.at[br], acc.at[dst_idx.at[bi]],
                                 ssem.at[br], add=True)

        def wait_scatter(j):
            bi, br = j % NBI, j % NBR
            _, valid = chunk_base(j)

            @pl.when(valid)
            def _():
                pltpu.make_async_copy(rows.at[br], acc.at[dst_idx.at[bi]],
                                      ssem.at[br]).wait()

        for m in range(NBI - 1):
            start_idx(m)
        pltpu.sync_copy(zeros_hbm, acc.at[pl.ds(t * ROW_STRIDE, ROW_SPAN)])
        plsc.subcore_barrier()
        for m in range(NBR - 1):
            wait_idx(m)
            start_gather(m)

        wait_idx(NBR - 1)
        start_gather(NBR - 1)
        wait_gather(0)
        start_scatter(0)
        start_idx(NBI - 1)

        def loop_body(j, carry):
            m = j + NBR - 1
            wait_idx(m)
            wait_scatter(j - 1)
            start_gather(m)
            wait_gather(j)
            start_scatter(j)
            start_idx(j + NBI - 1)
            return carry

        lax.fori_loop(1, N_ITERS, loop_body, 0)

        wait_scatter(N_ITERS - 1)
        for p in range(NBR - 1):
            wait_gather(N_ITERS + p)
        for p in range(NBI - NBR):
            wait_idx(N_ITERS + NBR - 1 + p)
        plsc.subcore_barrier()

        r0 = t * ROW_STRIDE

        @pl.when(c == 0)
        def _():
            pltpu.sync_copy(acc.at[pl.ds(r0, ROW_SPAN)],
                            part0_hbm.at[pl.ds(r0, ROW_SPAN)])

        @pl.when(c == 1)
        def _():
            pltpu.sync_copy(acc.at[pl.ds(r0, ROW_SPAN)],
                            part1_hbm.at[pl.ds(r0, ROW_SPAN)])

    return k


_sc_kernel = _sc_partials()


def _merge_body(a_ref, b_ref, o_ref):
    o_ref[...] = a_ref[...] + b_ref[...]


def kernel(x, edge_index):
    zeros = jnp.zeros((ROW_SPAN, D), jnp.float32)
    p0, p1 = _sc_kernel(x, edge_index, zeros)
    blk = 2000
    out = pl.pallas_call(
        _merge_body,
        out_shape=jax.ShapeDtypeStruct((N_NODES, D), jnp.float32),
        grid=(N_NODES // blk,),
        in_specs=[pl.BlockSpec((blk, D), lambda i: (i, 0))] * 2,
        out_specs=pl.BlockSpec((blk, D), lambda i: (i, 0)),
    )(p0, p1)
    return out


# speedup vs baseline: 14.4280x; 1.0007x over previous
import functools

import jax
import jax.numpy as jnp
from jax import lax
from jax.experimental import pallas as pl
from jax.experimental.pallas import tpu as pltpu
from jax.experimental.pallas import tpu_sc as plsc

N_NODES = 10000
N_EDGES = 320000
D = 128
NC = 2
NS = 16
CHUNK = 128

EDGES_PER_CORE = N_EDGES // NC
CHUNKS_PER_CORE = EDGES_PER_CORE // CHUNK
ROW_STRIDE = 624
ROW_SPAN = 640

NBI = 5
NBR = 2
N_ITERS = -(-CHUNKS_PER_CORE // NS)


def _sc_partials():
    mesh = plsc.VectorSubcoreMesh(core_axis_name="c", subcore_axis_name="s")

    @functools.partial(
        pl.kernel,
        out_type=(jax.ShapeDtypeStruct((N_NODES, D), jnp.float32),
                  jax.ShapeDtypeStruct((N_NODES, D), jnp.float32)),
        mesh=mesh,
        scratch_types=[
            pltpu.VMEM_SHARED((N_NODES, D), jnp.float32),
            pltpu.VMEM((NBI, CHUNK), jnp.int32),
            pltpu.VMEM((NBI, CHUNK), jnp.int32),
            pltpu.VMEM((NBR, CHUNK, D), jnp.float32),
            pltpu.SemaphoreType.DMA((NBI,)),
            pltpu.SemaphoreType.DMA((NBR,)),
            pltpu.SemaphoreType.DMA((NBR,)),
        ],
    )
    def k(x_hbm, ei_hbm, zeros_hbm, part0_hbm, part1_hbm,
          acc, src_idx, dst_idx, rows, isem, gsem, ssem):
        c = lax.axis_index("c")
        t = lax.axis_index("s")

        ebase = c * EDGES_PER_CORE

        def chunk_base(j):
            g = j * NS + t
            valid = g < CHUNKS_PER_CORE
            base = ebase + jnp.where(valid, g, 0) * CHUNK
            return base, valid

        def start_idx(j):
            bi = j % NBI
            base, _ = chunk_base(j)
            pltpu.async_copy(ei_hbm.at[1, pl.ds(base, CHUNK)], src_idx.at[bi],
                             isem.at[bi])
            pltpu.async_copy(ei_hbm.at[0, pl.ds(base, CHUNK)], dst_idx.at[bi],
                             isem.at[bi])

        def wait_idx(j):
            bi = j % NBI
            base, _ = chunk_base(j)
            pltpu.make_async_copy(ei_hbm.at[1, pl.ds(base, CHUNK)],
                                  src_idx.at[bi], isem.at[bi]).wait()
            pltpu.make_async_copy(ei_hbm.at[0, pl.ds(base, CHUNK)],
                                  dst_idx.at[bi], isem.at[bi]).wait()

        H = CHUNK // 2

        def start_gather(j):
            bi, br = j % NBI, j % NBR
            for h in range(2):
                pltpu.async_copy(x_hbm.at[src_idx.at[bi, pl.ds(h * H, H)]],
                                 rows.at[br, pl.ds(h * H, H)], gsem.at[br])

        def wait_gather(j):
            bi, br = j % NBI, j % NBR
            for h in range(2):
                pltpu.make_async_copy(x_hbm.at[src_idx.at[bi, pl.ds(h * H, H)]],
                                      rows.at[br, pl.ds(h * H, H)],
                                      gsem.at[br]).wait()

        def start_scatter(j):
            bi, br = j % NBI, j % NBR
            _, valid = chunk_base(j)

            @pl.when(valid)
            def _():
                pltpu.async_copy(rows.at[br], acc.at[dst_idx.at[bi]],
                                 ssem.at[br], add=True)

        def wait_scatter(j):
            bi, br = j % NBI, j % NBR
            _, valid = chunk_base(j)

            @pl.when(valid)
            def _():
                pltpu.make_async_copy(rows.at[br], acc.at[dst_idx.at[bi]],
                                      ssem.at[br]).wait()

        for m in range(NBI - 1):
            start_idx(m)
        pltpu.sync_copy(zeros_hbm, acc.at[pl.ds(t * ROW_STRIDE, ROW_SPAN)])
        plsc.subcore_barrier()
        for m in range(NBR - 1):
            wait_idx(m)
            start_gather(m)

        wait_idx(NBR - 1)
        start_gather(NBR - 1)
        wait_gather(0)
        start_scatter(0)
        start_idx(NBI - 1)

        def loop_body(j, carry):
            m = j + NBR - 1
            wait_idx(m)
            wait_scatter(j - 1)
            start_gather(m)
            wait_gather(j)
            start_scatter(j)
            start_idx(j + NBI - 1)
            return carry

        lax.fori_loop(1, N_ITERS, loop_body, 0)

        wait_scatter(N_ITERS - 1)
        for p in range(NBR - 1):
            wait_gather(N_ITERS + p)
        for p in range(NBI - NBR):
            wait_idx(N_ITERS + NBR - 1 + p)
        plsc.subcore_barrier()

        r0 = t * ROW_STRIDE

        @pl.when(c == 0)
        def _():
            pltpu.sync_copy(acc.at[pl.ds(r0, ROW_SPAN)],
                            part0_hbm.at[pl.ds(r0, ROW_SPAN)])

        @pl.when(c == 1)
        def _():
            pltpu.sync_copy(acc.at[pl.ds(r0, ROW_SPAN)],
                            part1_hbm.at[pl.ds(r0, ROW_SPAN)])

    return k


_sc_kernel = _sc_partials()


def _merge_body(a_ref, b_ref, o_ref):
    o_ref[...] = a_ref[...] + b_ref[...]


def kernel(x, edge_index):
    zeros = jnp.zeros((ROW_SPAN, D), jnp.float32)
    p0, p1 = _sc_kernel(x, edge_index, zeros)
    blk = 2000
    out = pl.pallas_call(
        _merge_body,
        out_shape=jax.ShapeDtypeStruct((N_NODES, D), jnp.float32),
        grid=(N_NODES // blk,),
        in_specs=[pl.BlockSpec((blk, D), lambda i: (i, 0))] * 2,
        out_specs=pl.BlockSpec((blk, D), lambda i: (i, 0)),
    )(p0, p1)
    return out


# speedup vs baseline: 14.5555x; 1.0088x over previous
import functools

import jax
import jax.numpy as jnp
from jax import lax
from jax.experimental import pallas as pl
from jax.experimental.pallas import tpu as pltpu
from jax.experimental.pallas import tpu_sc as plsc

N_NODES = 10000
N_EDGES = 320000
D = 128
NC = 2
NS = 16
CHUNK = 128

EDGES_PER_CORE = N_EDGES // NC
CHUNKS_PER_CORE = EDGES_PER_CORE // CHUNK
ROW_STRIDE = 624
ROW_SPAN = 640

NBI = 5
NBR = 2
N_ITERS = -(-CHUNKS_PER_CORE // NS)


def _sc_partials():
    mesh = plsc.VectorSubcoreMesh(core_axis_name="c", subcore_axis_name="s")

    @functools.partial(
        pl.kernel,
        out_type=(jax.ShapeDtypeStruct((N_NODES, D), jnp.float32),
                  jax.ShapeDtypeStruct((N_NODES, D), jnp.float32)),
        mesh=mesh,
        scratch_types=[
            pltpu.VMEM_SHARED((N_NODES, D), jnp.float32),
            pltpu.VMEM((NBI, CHUNK), jnp.int32),
            pltpu.VMEM((NBI, CHUNK), jnp.int32),
            pltpu.VMEM((NBR, CHUNK, D), jnp.float32),
            pltpu.SemaphoreType.DMA((NBI,)),
            pltpu.SemaphoreType.DMA((NBR,)),
            pltpu.SemaphoreType.DMA((NBR,)),
        ],
    )
    def k(x_hbm, ei_hbm, zeros_hbm, part0_hbm, part1_hbm,
          acc, src_idx, dst_idx, rows, isem, gsem, ssem):
        c = lax.axis_index("c")
        t = lax.axis_index("s")

        ebase = c * EDGES_PER_CORE

        def chunk_base(j):
            g = j * NS + t
            valid = g < CHUNKS_PER_CORE
            base = ebase + jnp.where(valid, g, 0) * CHUNK
            return base, valid

        def start_idx(j):
            bi = j % NBI
            base, _ = chunk_base(j)
            pltpu.async_copy(ei_hbm.at[1, pl.ds(base, CHUNK)], src_idx.at[bi],
                             isem.at[bi])
            pltpu.async_copy(ei_hbm.at[0, pl.ds(base, CHUNK)], dst_idx.at[bi],
                             isem.at[bi])

        def wait_idx(j):
            bi = j % NBI
            base, _ = chunk_base(j)
            pltpu.make_async_copy(ei_hbm.at[1, pl.ds(base, CHUNK)],
                                  src_idx.at[bi], isem.at[bi]).wait()
            pltpu.make_async_copy(ei_hbm.at[0, pl.ds(base, CHUNK)],
                                  dst_idx.at[bi], isem.at[bi]).wait()

        def start_gather(j):
            bi, br = j % NBI, j % NBR
            pltpu.async_copy(x_hbm.at[src_idx.at[bi]], rows.at[br], gsem.at[br])

        def wait_gather(j):
            bi, br = j % NBI, j % NBR
            pltpu.make_async_copy(x_hbm.at[src_idx.at[bi]], rows.at[br],
                                  gsem.at[br]).wait()

        def start_scatter(j):
            bi, br = j % NBI, j % NBR
            _, valid = chunk_base(j)

            @pl.when(valid)
            def _():
                pltpu.async_copy(rows.at[br], acc.at[dst_idx.at[bi]],
                                 ssem.at[br], add=True)

        def wait_scatter(j):
            bi, br = j % NBI, j % NBR
            _, valid = chunk_base(j)

            @pl.when(valid)
            def _():
                pltpu.make_async_copy(rows.at[br], acc.at[dst_idx.at[bi]],
                                      ssem.at[br]).wait()

        for m in range(NBI - 1):
            start_idx(m)
        pltpu.sync_copy(zeros_hbm, acc.at[pl.ds(t * ROW_STRIDE, ROW_SPAN)])
        plsc.subcore_barrier()
        for m in range(NBR - 1):
            wait_idx(m)
            start_gather(m)

        wait_idx(NBR - 1)
        start_gather(NBR - 1)
        wait_gather(0)
        start_scatter(0)
        start_idx(NBI - 1)

        def loop_body(j, carry):
            m = j + NBR - 1
            wait_idx(m)
            wait_scatter(j - 1)
            start_gather(m)
            wait_gather(j)
            start_scatter(j)
            start_idx(j + NBI - 1)
            return carry

        lax.fori_loop(1, N_ITERS, loop_body, 0)

        wait_scatter(N_ITERS - 1)
        for p in range(NBR - 1):
            wait_gather(N_ITERS + p)
        for p in range(NBI - NBR):
            wait_idx(N_ITERS + NBR - 1 + p)
        plsc.subcore_barrier()

        r0 = t * ROW_STRIDE

        @pl.when(c == 0)
        def _():
            pltpu.sync_copy(acc.at[pl.ds(r0, ROW_SPAN)],
                            part0_hbm.at[pl.ds(r0, ROW_SPAN)])

        @pl.when(c == 1)
        def _():
            pltpu.sync_copy(acc.at[pl.ds(r0, ROW_SPAN)],
                            part1_hbm.at[pl.ds(r0, ROW_SPAN)])

    return k


_sc_kernel = _sc_partials()


def _merge_body(a_ref, b_ref, o_ref):
    o_ref[...] = a_ref[...] + b_ref[...]


def kernel(x, edge_index):
    zeros = jnp.zeros((ROW_SPAN, D), jnp.float32)
    p0, p1 = _sc_kernel(x, edge_index, zeros)
    blk = 5000
    out = pl.pallas_call(
        _merge_body,
        out_shape=jax.ShapeDtypeStruct((N_NODES, D), jnp.float32),
        grid=(N_NODES // blk,),
        in_specs=[pl.BlockSpec((blk, D), lambda i: (i, 0))] * 2,
        out_specs=pl.BlockSpec((blk, D), lambda i: (i, 0)),
    )(p0, p1)
    return out


# speedup vs baseline: 14.6208x; 1.0045x over previous
import functools

import jax
import jax.numpy as jnp
from jax import lax
from jax.experimental import pallas as pl
from jax.experimental.pallas import tpu as pltpu
from jax.experimental.pallas import tpu_sc as plsc

N_NODES = 10000
N_EDGES = 320000
D = 128
NC = 2
NS = 16
CHUNK = 128

EDGES_PER_CORE = N_EDGES // NC
CHUNKS_PER_CORE = EDGES_PER_CORE // CHUNK
ROW_STRIDE = 624
ROW_SPAN = 640

NBI = 5
NBR = 2
N_ITERS = -(-CHUNKS_PER_CORE // NS)


def _sc_partials():
    mesh = plsc.VectorSubcoreMesh(core_axis_name="c", subcore_axis_name="s")

    @functools.partial(
        pl.kernel,
        out_type=(jax.ShapeDtypeStruct((N_NODES, D), jnp.float32),
                  jax.ShapeDtypeStruct((N_NODES, D), jnp.float32)),
        mesh=mesh,
        scratch_types=[
            pltpu.VMEM_SHARED((N_NODES, D), jnp.float32),
            pltpu.VMEM((NBI, CHUNK), jnp.int32),
            pltpu.VMEM((NBI, CHUNK), jnp.int32),
            pltpu.VMEM((NBR, CHUNK, D), jnp.float32),
            pltpu.SemaphoreType.DMA((NBI,)),
            pltpu.SemaphoreType.DMA((NBR,)),
            pltpu.SemaphoreType.DMA((NBR,)),
        ],
    )
    def k(x_hbm, ei_hbm, zeros_hbm, part0_hbm, part1_hbm,
          acc, src_idx, dst_idx, rows, isem, gsem, ssem):
        c = lax.axis_index("c")
        t = lax.axis_index("s")

        ebase = c * EDGES_PER_CORE

        def chunk_base(j):
            g = j * NS + t
            valid = g < CHUNKS_PER_CORE
            base = ebase + jnp.where(valid, g, 0) * CHUNK
            return base, valid

        def start_idx(j):
            bi = j % NBI
            base, _ = chunk_base(j)
            pltpu.async_copy(ei_hbm.at[1, pl.ds(base, CHUNK)], src_idx.at[bi],
                             isem.at[bi])
            pltpu.async_copy(ei_hbm.at[0, pl.ds(base, CHUNK)], dst_idx.at[bi],
                             isem.at[bi])

        def wait_idx(j):
            bi = j % NBI
            base, _ = chunk_base(j)
            pltpu.make_async_copy(ei_hbm.at[1, pl.ds(base, CHUNK)],
                                  src_idx.at[bi], isem.at[bi]).wait()
            pltpu.make_async_copy(ei_hbm.at[0, pl.ds(base, CHUNK)],
                                  dst_idx.at[bi], isem.at[bi]).wait()

        def start_gather(j):
            bi, br = j % NBI, j % NBR
            pltpu.async_copy(x_hbm.at[src_idx.at[bi]], rows.at[br], gsem.at[br])

        def wait_gather(j):
            bi, br = j % NBI, j % NBR
            pltpu.make_async_copy(x_hbm.at[src_idx.at[bi]], rows.at[br],
                                  gsem.at[br]).wait()

        def start_scatter(j):
            bi, br = j % NBI, j % NBR
            _, valid = chunk_base(j)

            @pl.when(valid)
            def _():
                pltpu.async_copy(rows.at[br], acc.at[dst_idx.at[bi]],
                                 ssem.at[br], priority=1, add=True)

        def wait_scatter(j):
            bi, br = j % NBI, j % NBR
            _, valid = chunk_base(j)

            @pl.when(valid)
            def _():
                pltpu.make_async_copy(rows.at[br], acc.at[dst_idx.at[bi]],
                                      ssem.at[br]).wait()

        for m in range(NBI - 1):
            start_idx(m)
        pltpu.sync_copy(zeros_hbm, acc.at[pl.ds(t * ROW_STRIDE, ROW_SPAN)])
        plsc.subcore_barrier()
        for m in range(NBR - 1):
            wait_idx(m)
            start_gather(m)

        wait_idx(NBR - 1)
        start_gather(NBR - 1)
        wait_gather(0)
        start_scatter(0)
        start_idx(NBI - 1)

        def loop_body(j, carry):
            m = j + NBR - 1
            wait_idx(m)
            wait_scatter(j - 1)
            start_gather(m)
            wait_gather(j)
            start_scatter(j)
            start_idx(j + NBI - 1)
            return carry

        lax.fori_loop(1, N_ITERS, loop_body, 0)

        wait_scatter(N_ITERS - 1)
        for p in range(NBR - 1):
            wait_gather(N_ITERS + p)
        for p in range(NBI - NBR):
            wait_idx(N_ITERS + NBR - 1 + p)
        plsc.subcore_barrier()

        r0 = t * ROW_STRIDE

        @pl.when(c == 0)
        def _():
            pltpu.sync_copy(acc.at[pl.ds(r0, ROW_SPAN)],
                            part0_hbm.at[pl.ds(r0, ROW_SPAN)])

        @pl.when(c == 1)
        def _():
            pltpu.sync_copy(acc.at[pl.ds(r0, ROW_SPAN)],
                            part1_hbm.at[pl.ds(r0, ROW_SPAN)])

    return k


_sc_kernel = _sc_partials()


def _merge_body(a_ref, b_ref, o_ref):
    o_ref[...] = a_ref[...] + b_ref[...]


def kernel(x, edge_index):
    zeros = jnp.zeros((ROW_SPAN, D), jnp.float32)
    p0, p1 = _sc_kernel(x, edge_index, zeros)
    blk = 5000
    out = pl.pallas_call(
        _merge_body,
        out_shape=jax.ShapeDtypeStruct((N_NODES, D), jnp.float32),
        grid=(N_NODES // blk,),
        in_specs=[pl.BlockSpec((blk, D), lambda i: (i, 0))] * 2,
        out_specs=pl.BlockSpec((blk, D), lambda i: (i, 0)),
    )(p0, p1)
    return out
